# baseline (speedup 1.0000x reference)
import functools

import jax
import jax.numpy as jnp
from jax import lax
from jax.experimental import pallas as pl
from jax.experimental.pallas import tpu as pltpu
from jax.experimental.pallas import tpu_sc as plsc

E = 8
K = 2
D = 768
H = 2048
N = 2048
BLK = 256
NB = N * K // BLK + E - 1
NB = 24
NSLOT = NB * BLK
NW = 32
TPW = N // NW


def _route_kernel(x_ref, gate_w_ref, gate_b_ref,
                  pos_ref, cmap_ref, rexp_ref, wl_ref):
    xb = x_ref[...]
    logits = jnp.dot(xb.astype(jnp.bfloat16),
                     gate_w_ref[...].astype(jnp.bfloat16),
                     preferred_element_type=jnp.float32) + gate_b_ref[...]
    eio = lax.broadcasted_iota(jnp.int32, (N, E), 1)
    m1 = jnp.max(logits, axis=1, keepdims=True)
    a1 = jnp.min(jnp.where(logits == m1, eio, E), axis=1, keepdims=True)
    lm = jnp.where(eio == a1, -jnp.inf, logits)
    m2 = jnp.max(lm, axis=1, keepdims=True)
    a2 = jnp.min(jnp.where(lm == m2, eio, E), axis=1, keepdims=True)
    t = jnp.exp(m2 - m1)
    w1 = 1.0 / (1.0 + t)
    w2 = t / (1.0 + t)

    oh = (jnp.where(eio == a1, 1.0, 0.0) + jnp.where(eio == a2, 1.0, 0.0))

    tio_r = lax.broadcasted_iota(jnp.int32, (N, N), 0)
    tio_c = lax.broadcasted_iota(jnp.int32, (N, N), 1)
    ltri = jnp.where(tio_c < tio_r, 1.0, 0.0).astype(jnp.bfloat16)
    pcum = jnp.dot(ltri, oh.astype(jnp.bfloat16),
                   preferred_element_type=jnp.float32)

    counts = jnp.sum(oh, axis=0, keepdims=True)
    padded = jnp.ceil(counts * (1.0 / BLK)) * BLK
    eio8_r = lax.broadcasted_iota(jnp.int32, (E, E), 0)
    eio8_c = lax.broadcasted_iota(jnp.int32, (E, E), 1)
    sut = jnp.where(eio8_r < eio8_c, 1.0, 0.0).astype(jnp.bfloat16)
    starts = jnp.dot(padded.astype(jnp.bfloat16), sut,
                     preferred_element_type=jnp.float32)

    starts_b = jnp.broadcast_to(starts, (N, E))
    base = starts_b + pcum
    pos1 = jnp.sum(jnp.where(eio == a1, base, 0.0), axis=1, keepdims=True)
    pos2 = jnp.sum(jnp.where(eio == a2, base, 0.0), axis=1, keepdims=True)
    pos_ref[...] = jnp.concatenate([pos1, pos2], axis=1).astype(jnp.int32)
    wl_ref[0] = jnp.broadcast_to(w1, (N, 16))
    wl_ref[1] = jnp.broadcast_to(w2, (N, 16))

    bio = lax.broadcasted_iota(jnp.int32, (128, E), 0).astype(jnp.float32) * float(BLK)
    s_b = jnp.broadcast_to(starts, (128, E))
    p_b = jnp.broadcast_to(padded, (128, E))
    inr = jnp.where((bio >= s_b) & (bio < s_b + p_b), 1.0, 0.0)
    eval_ = lax.broadcasted_iota(jnp.int32, (128, E), 1).astype(jnp.float32)
    used = jnp.sum(inr, axis=1, keepdims=True)
    uexp = jnp.where(padded > 0.0, 1.0, 0.0)
    rank = jnp.dot(uexp.astype(jnp.bfloat16), sut,
                   preferred_element_type=jnp.float32)
    nrun = jnp.sum(uexp, axis=1, keepdims=True)
    crun = jnp.sum(inr * jnp.broadcast_to(rank, (128, E)), axis=1,
                   keepdims=True)
    cmap_ref[...] = jnp.where(used > 0.0, crun,
                              jnp.broadcast_to(nrun, (128, 1)) - 1.0
                              ).astype(jnp.int32)
    rio = lax.broadcasted_iota(jnp.int32, (16, E), 0).astype(jnp.float32)
    rsel = jnp.where((jnp.broadcast_to(rank, (16, E)) == rio)
                     & (jnp.broadcast_to(uexp, (16, E)) > 0.0), 1.0, 0.0)
    rexp_ref[...] = jnp.sum(
        rsel * lax.broadcasted_iota(jnp.int32, (16, E), 1).astype(jnp.float32),
        axis=1, keepdims=True).astype(jnp.int32)


def _route(xs, gate_w, gate_b):
    return pl.pallas_call(
        _route_kernel,
        in_specs=[
            pl.BlockSpec((N, D), lambda: (0, 0)),
            pl.BlockSpec((D, E), lambda: (0, 0)),
            pl.BlockSpec((1, E), lambda: (0, 0)),
        ],
        out_specs=[
            pl.BlockSpec((N, K), lambda: (0, 0)),
            pl.BlockSpec((128, 1), lambda: (0, 0)),
            pl.BlockSpec((16, 1), lambda: (0, 0)),
            pl.BlockSpec((K, N, 16), lambda: (0, 0, 0)),
        ],
        out_shape=[
            jax.ShapeDtypeStruct((N, K), jnp.int32),
            jax.ShapeDtypeStruct((128, 1), jnp.int32),
            jax.ShapeDtypeStruct((16, 1), jnp.int32),
            jax.ShapeDtypeStruct((K, N, 16), jnp.float32),
        ],
    )(xs, gate_w, gate_b.reshape(1, E))


def _sc_mesh():
    return plsc.VectorSubcoreMesh(core_axis_name="c", subcore_axis_name="s")


@jax.jit
def _dispatch(xs, pos_t):
    @functools.partial(
        pl.kernel,
        out_type=jax.ShapeDtypeStruct((NSLOT, D), jnp.float32),
        mesh=_sc_mesh(),
        scratch_types=[
            pltpu.VMEM((TPW,), jnp.int32),
            pltpu.VMEM((TPW,), jnp.int32),
            pltpu.VMEM((TPW, D), jnp.float32),
            pltpu.SemaphoreType.DMA,
        ],
    )
    def k(x_hbm, pos_hbm, xg_hbm, i0_v, i1_v, rows_v, sem):
        wid = lax.axis_index("s") * 2 + lax.axis_index("c")
        base = wid * TPW
        pltpu.sync_copy(pos_hbm.at[0, pl.ds(base, TPW)], i0_v)
        pltpu.sync_copy(pos_hbm.at[1, pl.ds(base, TPW)], i1_v)
        pltpu.async_copy(x_hbm.at[pl.ds(base, TPW)], rows_v, sem).wait()
        pltpu.async_copy(rows_v, xg_hbm.at[i0_v], sem).wait()
        pltpu.async_copy(rows_v, xg_hbm.at[i1_v], sem).wait()

    return k(xs, pos_t)


def _ffn_kernel(cmap_ref, rexp_ref, xg_ref, bg_ref, b1_ref, b2_ref,
                Wg_hbm, W1_hbm, W2_hbm, y_ref, wgb, w1b, w2b, sems):
    b = pl.program_id(0)
    r = cmap_ref[b]
    slot = lax.rem(r, 2)
    prev_r = cmap_ref[jnp.maximum(b - 1, 0)]
    first_of_run = jnp.logical_or(b == 0, r != prev_r)
    nrun = cmap_ref[NB - 1] + 1

    def issue(run, s):
        e = rexp_ref[run]
        pltpu.make_async_copy(Wg_hbm.at[e], wgb.at[s], sems.at[s, 0]).start()
        pltpu.make_async_copy(W1_hbm.at[e], w1b.at[s], sems.at[s, 1]).start()
        pltpu.make_async_copy(W2_hbm.at[e], w2b.at[s], sems.at[s, 2]).start()

    @pl.when(b == 0)
    def _prime():
        issue(0, 0)

    @pl.when(first_of_run)
    def _swap():
        pltpu.make_async_copy(Wg_hbm.at[0], wgb.at[slot], sems.at[slot, 0]).wait()
        pltpu.make_async_copy(W1_hbm.at[0], w1b.at[slot], sems.at[slot, 1]).wait()
        pltpu.make_async_copy(W2_hbm.at[0], w2b.at[slot], sems.at[slot, 2]).wait()

        @pl.when(r + 1 < nrun)
        def _prefetch():
            issue(r + 1, 1 - slot)

    xb16 = xg_ref[...].astype(jnp.bfloat16)
    g = jnp.dot(xb16, wgb[slot].astype(jnp.bfloat16),
                preferred_element_type=jnp.float32) + bg_ref[0]
    g = g * lax.logistic(g)
    u = jnp.dot(xb16, w1b[slot].astype(jnp.bfloat16),
                preferred_element_type=jnp.float32) + b1_ref[0]
    hid = (g * u).astype(jnp.bfloat16)
    y_ref[...] = jnp.dot(hid, w2b[slot].astype(jnp.bfloat16),
                         preferred_element_type=jnp.float32) + b2_ref[0]


def _ffn(cmap, rexp, xg, Wg, bg, W1, b1, W2, b2):
    grid_spec = pltpu.PrefetchScalarGridSpec(
        num_scalar_prefetch=2,
        grid=(NB,),
        in_specs=[
            pl.BlockSpec((BLK, D), lambda b, c, rx: (b, 0)),
            pl.BlockSpec((1, 1, H), lambda b, c, rx: (rx[c[b]], 0, 0)),
            pl.BlockSpec((1, 1, H), lambda b, c, rx: (rx[c[b]], 0, 0)),
            pl.BlockSpec((1, 1, D), lambda b, c, rx: (rx[c[b]], 0, 0)),
            pl.BlockSpec(memory_space=pl.ANY),
            pl.BlockSpec(memory_space=pl.ANY),
            pl.BlockSpec(memory_space=pl.ANY),
        ],
        out_specs=pl.BlockSpec((BLK, D), lambda b, c, rx: (b, 0)),
        scratch_shapes=[
            pltpu.VMEM((2, D, H), jnp.float32),
            pltpu.VMEM((2, D, H), jnp.float32),
            pltpu.VMEM((2, H, D), jnp.float32),
            pltpu.SemaphoreType.DMA((2, 3)),
        ],
    )
    return pl.pallas_call(
        _ffn_kernel,
        grid_spec=grid_spec,
        out_shape=jax.ShapeDtypeStruct((NSLOT, D), jnp.float32),
        compiler_params=pltpu.CompilerParams(
            dimension_semantics=("arbitrary",),
        ),
    )(cmap, rexp, xg, bg.reshape(E, 1, H), b1.reshape(E, 1, H),
      b2.reshape(E, 1, D), Wg, W1, W2)


@jax.jit
def _combine(y, pos_t, wl):
    @functools.partial(
        pl.kernel,
        out_type=jax.ShapeDtypeStruct((N, D), jnp.float32),
        mesh=_sc_mesh(),
        scratch_types=[
            pltpu.VMEM((TPW,), jnp.int32),
            pltpu.VMEM((TPW,), jnp.int32),
            pltpu.VMEM((TPW, 16), jnp.float32),
            pltpu.VMEM((TPW, 16), jnp.float32),
            pltpu.VMEM((TPW, D), jnp.float32),
            pltpu.VMEM((TPW, D), jnp.float32),
            pltpu.SemaphoreType.DMA,
        ],
    )
    def k(y_hbm, pos_hbm, wl_hbm, out_hbm, i0_v, i1_v, w0_v, w1_v,
          a_v, b_v, sem):
        wid = lax.axis_index("s") * 2 + lax.axis_index("c")
        base = wid * TPW
        pltpu.sync_copy(pos_hbm.at[0, pl.ds(base, TPW)], i0_v)
        pltpu.sync_copy(pos_hbm.at[1, pl.ds(base, TPW)], i1_v)
        pltpu.sync_copy(wl_hbm.at[0, pl.ds(base, TPW)], w0_v)
        pltpu.sync_copy(wl_hbm.at[1, pl.ds(base, TPW)], w1_v)
        cp_a = pltpu.async_copy(y_hbm.at[i0_v], a_v, sem)
        cp_b = pltpu.async_copy(y_hbm.at[i1_v], b_v, sem)
        cp_a.wait()
        cp_b.wait()

        @pl.loop(0, TPW)
        def _(i):
            w0 = w0_v[i, :]
            w1 = w1_v[i, :]

            @pl.loop(0, D, step=16)
            def _(j):
                a_v[i, pl.ds(j, 16)] = (w0 * a_v[i, pl.ds(j, 16)]
                                        + w1 * b_v[i, pl.ds(j, 16)])

        pltpu.sync_copy(a_v, out_hbm.at[pl.ds(base, TPW)])

    return k(y, pos_t, wl)


def kernel(x, gate_w, gate_b, Wg, bg, W1, b1, W2, b2):
    xs = x.reshape(N, D)
    pos, cmap2d, rexp2d, wl = _route(xs, gate_w, gate_b)
    pos_t = pos.T
    cmap = cmap2d.reshape(128)[:NB]
    rexp = rexp2d.reshape(16)
    xg = _dispatch(xs, pos_t)
    y = _ffn(cmap, rexp, xg, Wg, bg, W1, b1, W2, b2)
    out = _combine(y, pos_t, wl)
    return out.reshape(x.shape)


# speedup vs baseline: 1.0617x; 1.0617x over previous
import functools

import jax
import jax.numpy as jnp
from jax import lax
from jax.experimental import pallas as pl
from jax.experimental.pallas import tpu as pltpu
from jax.experimental.pallas import tpu_sc as plsc

E = 8
K = 2
D = 768
H = 2048
N = 2048
BLK = 256
NB = N * K // BLK + E - 1
NB = 24
NSLOT = NB * BLK
NW = 32
TPW = N // NW


def _route_kernel(x_ref, gate_w_ref, gate_b_ref,
                  pos_ref, w_ref, cmap_ref, rexp_ref):
    xb = x_ref[...]
    logits = jnp.dot(xb.astype(jnp.bfloat16),
                     gate_w_ref[...].astype(jnp.bfloat16),
                     preferred_element_type=jnp.float32) + gate_b_ref[...]
    eio = lax.broadcasted_iota(jnp.int32, (N, E), 1)
    m1 = jnp.max(logits, axis=1, keepdims=True)
    a1 = jnp.min(jnp.where(logits == m1, eio, E), axis=1, keepdims=True)
    lm = jnp.where(eio == a1, -jnp.inf, logits)
    m2 = jnp.max(lm, axis=1, keepdims=True)
    a2 = jnp.min(jnp.where(lm == m2, eio, E), axis=1, keepdims=True)
    t = jnp.exp(m2 - m1)
    w1 = 1.0 / (1.0 + t)
    w2 = t / (1.0 + t)

    oh = (jnp.where(eio == a1, 1.0, 0.0) + jnp.where(eio == a2, 1.0, 0.0))

    tio_r = lax.broadcasted_iota(jnp.int32, (N, N), 0)
    tio_c = lax.broadcasted_iota(jnp.int32, (N, N), 1)
    ltri = jnp.where(tio_c < tio_r, 1.0, 0.0).astype(jnp.bfloat16)
    pcum = jnp.dot(ltri, oh.astype(jnp.bfloat16),
                   preferred_element_type=jnp.float32)

    counts = jnp.sum(oh, axis=0, keepdims=True)
    padded = jnp.ceil(counts * (1.0 / BLK)) * BLK
    eio8_r = lax.broadcasted_iota(jnp.int32, (E, E), 0)
    eio8_c = lax.broadcasted_iota(jnp.int32, (E, E), 1)
    sut = jnp.where(eio8_r < eio8_c, 1.0, 0.0).astype(jnp.bfloat16)
    starts = jnp.dot(padded.astype(jnp.bfloat16), sut,
                     preferred_element_type=jnp.float32)

    starts_b = jnp.broadcast_to(starts, (N, E))
    base = starts_b + pcum
    pos1 = jnp.sum(jnp.where(eio == a1, base, 0.0), axis=1, keepdims=True)
    pos2 = jnp.sum(jnp.where(eio == a2, base, 0.0), axis=1, keepdims=True)
    pos_ref[...] = jnp.concatenate([pos1, pos2], axis=1).astype(jnp.int32)
    w_ref[...] = jnp.concatenate([w1, w2], axis=1)

    bio = lax.broadcasted_iota(jnp.int32, (128, E), 0).astype(jnp.float32) * float(BLK)
    s_b = jnp.broadcast_to(starts, (128, E))
    p_b = jnp.broadcast_to(padded, (128, E))
    inr = jnp.where((bio >= s_b) & (bio < s_b + p_b), 1.0, 0.0)
    eval_ = lax.broadcasted_iota(jnp.int32, (128, E), 1).astype(jnp.float32)
    used = jnp.sum(inr, axis=1, keepdims=True)
    uexp = jnp.where(padded > 0.0, 1.0, 0.0)
    rank = jnp.dot(uexp.astype(jnp.bfloat16), sut,
                   preferred_element_type=jnp.float32)
    nrun = jnp.sum(uexp, axis=1, keepdims=True)
    crun = jnp.sum(inr * jnp.broadcast_to(rank, (128, E)), axis=1,
                   keepdims=True)
    cmap_ref[...] = jnp.where(used > 0.0, crun,
                              jnp.broadcast_to(nrun, (128, 1)) - 1.0
                              ).astype(jnp.int32)
    rio = lax.broadcasted_iota(jnp.int32, (16, E), 0).astype(jnp.float32)
    rsel = jnp.where((jnp.broadcast_to(rank, (16, E)) == rio)
                     & (jnp.broadcast_to(uexp, (16, E)) > 0.0), 1.0, 0.0)
    rexp_ref[...] = jnp.sum(
        rsel * lax.broadcasted_iota(jnp.int32, (16, E), 1).astype(jnp.float32),
        axis=1, keepdims=True).astype(jnp.int32)


def _route(xs, gate_w, gate_b):
    return pl.pallas_call(
        _route_kernel,
        in_specs=[
            pl.BlockSpec((N, D), lambda: (0, 0)),
            pl.BlockSpec((D, E), lambda: (0, 0)),
            pl.BlockSpec((1, E), lambda: (0, 0)),
        ],
        out_specs=[
            pl.BlockSpec((N, K), lambda: (0, 0)),
            pl.BlockSpec((N, K), lambda: (0, 0)),
            pl.BlockSpec((128, 1), lambda: (0, 0)),
            pl.BlockSpec((16, 1), lambda: (0, 0)),
        ],
        out_shape=[
            jax.ShapeDtypeStruct((N, K), jnp.int32),
            jax.ShapeDtypeStruct((N, K), jnp.float32),
            jax.ShapeDtypeStruct((128, 1), jnp.int32),
            jax.ShapeDtypeStruct((16, 1), jnp.int32),
        ],
    )(xs, gate_w, gate_b.reshape(1, E))


def _sc_mesh():
    return plsc.VectorSubcoreMesh(core_axis_name="c", subcore_axis_name="s")


@jax.jit
def _dispatch(xs, pos_t):
    @functools.partial(
        pl.kernel,
        out_type=jax.ShapeDtypeStruct((NSLOT, D), jnp.float32),
        mesh=_sc_mesh(),
        scratch_types=[
            pltpu.VMEM((TPW,), jnp.int32),
            pltpu.VMEM((TPW,), jnp.int32),
            pltpu.VMEM((TPW, D), jnp.float32),
            pltpu.SemaphoreType.DMA,
        ],
    )
    def k(x_hbm, pos_hbm, xg_hbm, i0_v, i1_v, rows_v, sem):
        wid = lax.axis_index("s") * 2 + lax.axis_index("c")
        base = wid * TPW
        pltpu.sync_copy(pos_hbm.at[0, pl.ds(base, TPW)], i0_v)
        pltpu.sync_copy(pos_hbm.at[1, pl.ds(base, TPW)], i1_v)
        pltpu.async_copy(x_hbm.at[pl.ds(base, TPW)], rows_v, sem).wait()
        pltpu.async_copy(rows_v, xg_hbm.at[i0_v], sem).wait()
        pltpu.async_copy(rows_v, xg_hbm.at[i1_v], sem).wait()

    return k(xs, pos_t)


def _ffn_kernel(cmap_ref, rexp_ref, xg_ref, bg_ref, b1_ref, b2_ref,
                Wg_hbm, W1_hbm, W2_hbm, y_ref, wgb, w1b, w2b, sems):
    b = pl.program_id(0)
    r = cmap_ref[b]
    slot = lax.rem(r, 2)
    prev_r = cmap_ref[jnp.maximum(b - 1, 0)]
    first_of_run = jnp.logical_or(b == 0, r != prev_r)
    nrun = cmap_ref[NB - 1] + 1

    def issue(run, s):
        e = rexp_ref[run]
        pltpu.make_async_copy(Wg_hbm.at[e], wgb.at[s], sems.at[s, 0]).start()
        pltpu.make_async_copy(W1_hbm.at[e], w1b.at[s], sems.at[s, 1]).start()
        pltpu.make_async_copy(W2_hbm.at[e], w2b.at[s], sems.at[s, 2]).start()

    @pl.when(b == 0)
    def _prime():
        issue(0, 0)

    @pl.when(first_of_run)
    def _swap():
        pltpu.make_async_copy(Wg_hbm.at[0], wgb.at[slot], sems.at[slot, 0]).wait()
        pltpu.make_async_copy(W1_hbm.at[0], w1b.at[slot], sems.at[slot, 1]).wait()
        pltpu.make_async_copy(W2_hbm.at[0], w2b.at[slot], sems.at[slot, 2]).wait()

        @pl.when(r + 1 < nrun)
        def _prefetch():
            issue(r + 1, 1 - slot)

    xb16 = xg_ref[...].astype(jnp.bfloat16)

    def compute(s):
        g = jnp.dot(xb16, wgb[s].astype(jnp.bfloat16),
                    preferred_element_type=jnp.float32) + bg_ref[0]
        g = g * lax.logistic(g)
        u = jnp.dot(xb16, w1b[s].astype(jnp.bfloat16),
                    preferred_element_type=jnp.float32) + b1_ref[0]
        hid = (g * u).astype(jnp.bfloat16)
        y_ref[...] = jnp.dot(hid, w2b[s].astype(jnp.bfloat16),
                             preferred_element_type=jnp.float32) + b2_ref[0]

    @pl.when(slot == 0)
    def _s0():
        compute(0)

    @pl.when(slot == 1)
    def _s1():
        compute(1)


def _ffn(cmap, rexp, xg, Wg, bg, W1, b1, W2, b2):
    grid_spec = pltpu.PrefetchScalarGridSpec(
        num_scalar_prefetch=2,
        grid=(NB,),
        in_specs=[
            pl.BlockSpec((BLK, D), lambda b, c, rx: (b, 0)),
            pl.BlockSpec((1, 1, H), lambda b, c, rx: (rx[c[b]], 0, 0)),
            pl.BlockSpec((1, 1, H), lambda b, c, rx: (rx[c[b]], 0, 0)),
            pl.BlockSpec((1, 1, D), lambda b, c, rx: (rx[c[b]], 0, 0)),
            pl.BlockSpec(memory_space=pl.ANY),
            pl.BlockSpec(memory_space=pl.ANY),
            pl.BlockSpec(memory_space=pl.ANY),
        ],
        out_specs=pl.BlockSpec((BLK, D), lambda b, c, rx: (b, 0)),
        scratch_shapes=[
            pltpu.VMEM((2, D, H), jnp.float32),
            pltpu.VMEM((2, D, H), jnp.float32),
            pltpu.VMEM((2, H, D), jnp.float32),
            pltpu.SemaphoreType.DMA((2, 3)),
        ],
    )
    return pl.pallas_call(
        _ffn_kernel,
        grid_spec=grid_spec,
        out_shape=jax.ShapeDtypeStruct((NSLOT, D), jnp.float32),
        compiler_params=pltpu.CompilerParams(
            dimension_semantics=("arbitrary",),
        ),
    )(cmap, rexp, xg, bg.reshape(E, 1, H), b1.reshape(E, 1, H),
      b2.reshape(E, 1, D), Wg, W1, W2)


@jax.jit
def _combine_gather(y, pos_t):
    @functools.partial(
        pl.kernel,
        out_type=[
            jax.ShapeDtypeStruct((N, D), jnp.float32),
            jax.ShapeDtypeStruct((N, D), jnp.float32),
        ],
        mesh=_sc_mesh(),
        scratch_types=[
            pltpu.VMEM((TPW,), jnp.int32),
            pltpu.VMEM((TPW,), jnp.int32),
            pltpu.VMEM((TPW, D), jnp.float32),
            pltpu.VMEM((TPW, D), jnp.float32),
            pltpu.SemaphoreType.DMA,
        ],
    )
    def k(y_hbm, pos_hbm, a_hbm, b_hbm, i0_v, i1_v, a_v, b_v, sem):
        wid = lax.axis_index("s") * 2 + lax.axis_index("c")
        base = wid * TPW
        pltpu.sync_copy(pos_hbm.at[0, pl.ds(base, TPW)], i0_v)
        pltpu.sync_copy(pos_hbm.at[1, pl.ds(base, TPW)], i1_v)
        pltpu.async_copy(y_hbm.at[i0_v], a_v, sem).wait()
        pltpu.async_copy(y_hbm.at[i1_v], b_v, sem).wait()
        pltpu.sync_copy(a_v, a_hbm.at[pl.ds(base, TPW)])
        pltpu.sync_copy(b_v, b_hbm.at[pl.ds(base, TPW)])

    return k(y, pos_t)


def _mix_kernel(w_ref, a_ref, b_ref, out_ref):
    w = w_ref[...]
    out_ref[...] = w[:, 0:1] * a_ref[...] + w[:, 1:2] * b_ref[...]


def _mix(wts, a, b):
    return pl.pallas_call(
        _mix_kernel,
        grid=(N // 512,),
        in_specs=[
            pl.BlockSpec((512, K), lambda i: (i, 0)),
            pl.BlockSpec((512, D), lambda i: (i, 0)),
            pl.BlockSpec((512, D), lambda i: (i, 0)),
        ],
        out_specs=pl.BlockSpec((512, D), lambda i: (i, 0)),
        out_shape=jax.ShapeDtypeStruct((N, D), jnp.float32),
    )(wts, a, b)


def kernel(x, gate_w, gate_b, Wg, bg, W1, b1, W2, b2):
    xs = x.reshape(N, D)
    pos, wts, cmap2d, rexp2d = _route(xs, gate_w, gate_b)
    pos_t = pos.T
    cmap = cmap2d.reshape(128)[:NB]
    rexp = rexp2d.reshape(16)
    xg = _dispatch(xs, pos_t)
    y = _ffn(cmap, rexp, xg, Wg, bg, W1, b1, W2, b2)
    a, b = _combine_gather(y, pos_t)
    out = _mix(wts, a, b)
    return out.reshape(x.shape)
